# baseline (speedup 1.0000x reference)
import functools

import jax
import jax.numpy as jnp
from jax import lax
from jax.experimental import pallas as pl
from jax.experimental.pallas import tpu as pltpu
from jax.experimental.pallas import tpu_sc as plsc

HIDDEN = 768
BATCH = 4
SEQ = 2048
NC = 2
NS = 16
NW = NC * NS
TOTAL = BATCH * SEQ
RPW = TOTAL // NW
CHUNK = 32
NCHUNK = RPW // CHUNK
LANES = 16
NSLICE = HIDDEN // LANES


def _emb_body(idx_hbm, pos_hbm, tab_hbm, out_hbm, idx_v, rows_v, pos_v,
              gsem0, gsem1, psem0, psem1, osem0, osem1):
    wid = lax.axis_index("s") * NC + lax.axis_index("c")
    base = wid * RPW
    pos_base = lax.rem(base, SEQ)
    pltpu.sync_copy(idx_hbm.at[wid], idx_v)

    gsem = (gsem0, gsem1)
    psem = (psem0, psem1)
    osem = (osem0, osem1)
    gd = [None, None]
    pd = [None, None]
    od = [None, None]

    def start_in(c):
        b = c & 1
        gd[b] = pltpu.async_copy(tab_hbm.at[idx_v.at[c]], rows_v.at[b], gsem[b])
        pd[b] = pltpu.async_copy(
            pos_hbm.at[pl.ds(pos_base + c * CHUNK, CHUNK)], pos_v.at[b], psem[b])

    start_in(0)
    for c in range(NCHUNK):
        b = c & 1
        if c + 1 < NCHUNK:
            if c >= 1:
                od[1 - b].wait()
            start_in(c + 1)
        gd[b].wait()
        pd[b].wait()

        def add_row(r, carry):
            for j in range(NSLICE):
                sl = pl.ds(j * LANES, LANES)
                plsc.addupdate(rows_v.at[b, r, sl], pos_v[b, r, sl])
            return carry

        lax.fori_loop(0, CHUNK, add_row, 0)
        od[b] = pltpu.async_copy(
            rows_v.at[b], out_hbm.at[pl.ds(base + c * CHUNK, CHUNK)], osem[b])
    od[0].wait()
    od[1].wait()


@jax.jit
def _emb(idx, token_table, pos_table):
    mesh = plsc.VectorSubcoreMesh(core_axis_name="c", subcore_axis_name="s")
    f = pl.kernel(
        _emb_body,
        mesh=mesh,
        out_type=jax.ShapeDtypeStruct((TOTAL, HIDDEN), jnp.float32),
        scratch_types=[
            pltpu.VMEM((NCHUNK, CHUNK), jnp.int32),
            pltpu.VMEM((2, CHUNK, HIDDEN), jnp.float32),
            pltpu.VMEM((2, CHUNK, HIDDEN), jnp.float32),
            pltpu.SemaphoreType.DMA,
            pltpu.SemaphoreType.DMA,
            pltpu.SemaphoreType.DMA,
            pltpu.SemaphoreType.DMA,
            pltpu.SemaphoreType.DMA,
            pltpu.SemaphoreType.DMA,
        ],
    )
    return f(idx, pos_table, token_table)


def kernel(input_ids, token_table, pos_table):
    idx = input_ids.reshape(NW, NCHUNK, CHUNK).astype(jnp.int32)
    out = _emb(idx, token_table, pos_table)
    return out.reshape(BATCH, SEQ, HIDDEN)


# speedup vs baseline: 1.1921x; 1.1921x over previous
import functools

import jax
import jax.numpy as jnp
from jax import lax
from jax.experimental import pallas as pl
from jax.experimental.pallas import tpu as pltpu
from jax.experimental.pallas import tpu_sc as plsc

HIDDEN = 768
BATCH = 4
SEQ = 2048
NC = 2
NS = 16
NW = NC * NS
TOTAL = BATCH * SEQ
RPW = TOTAL // NW
CHUNK = 32
NCHUNK = RPW // CHUNK
LANES = 16
NSLICE = HIDDEN // LANES


def _emb_body(idx_hbm, pos_hbm, tab_hbm, out_hbm, idx_v, rows_v, pos_v,
              pos_sh, gsem0, gsem1, psem, osem0, osem1):
    sid = lax.axis_index("s")
    cc = lax.axis_index("c")
    wid = sid * NC + cc
    base = wid * RPW
    pltpu.sync_copy(idx_hbm.at[wid], idx_v)

    stage_src = (2 * (sid // 4) + cc) * 256 + lax.rem(sid, 4) * 64
    pltpu.sync_copy(pos_hbm.at[pl.ds(stage_src, 64)],
                    pos_sh.at[pl.ds(sid * 64, 64)])
    plsc.subcore_barrier()
    pos_base = lax.rem(sid, 4) * 256

    gsem = (gsem0, gsem1)
    osem = (osem0, osem1)
    gd = [None, None]
    od = [None, None]

    def start_gather(c):
        b = c & 1
        gd[b] = pltpu.async_copy(tab_hbm.at[idx_v.at[c]], rows_v.at[b], gsem[b])

    start_gather(0)
    for c in range(NCHUNK):
        b = c & 1
        pd = pltpu.async_copy(
            pos_sh.at[pl.ds(pos_base + c * CHUNK, CHUNK)], pos_v, psem)
        if c + 1 < NCHUNK:
            if c >= 1:
                od[1 - b].wait()
            start_gather(c + 1)
        gd[b].wait()
        pd.wait()

        def add_row(r, carry):
            for j in range(NSLICE):
                sl = pl.ds(j * LANES, LANES)
                plsc.addupdate(rows_v.at[b, r, sl], pos_v[r, sl])
            return carry

        lax.fori_loop(0, CHUNK, add_row, 0)
        od[b] = pltpu.async_copy(
            rows_v.at[b], out_hbm.at[pl.ds(base + c * CHUNK, CHUNK)], osem[b])
    od[0].wait()
    od[1].wait()


@jax.jit
def _emb(idx, token_table, pos_table):
    mesh = plsc.VectorSubcoreMesh(core_axis_name="c", subcore_axis_name="s")
    f = pl.kernel(
        _emb_body,
        mesh=mesh,
        out_type=jax.ShapeDtypeStruct((TOTAL, HIDDEN), jnp.float32),
        scratch_types=[
            pltpu.VMEM((NCHUNK, CHUNK), jnp.int32),
            pltpu.VMEM((2, CHUNK, HIDDEN), jnp.float32),
            pltpu.VMEM((CHUNK, HIDDEN), jnp.float32),
            pltpu.VMEM_SHARED((4 * 256, HIDDEN), jnp.float32),
            pltpu.SemaphoreType.DMA,
            pltpu.SemaphoreType.DMA,
            pltpu.SemaphoreType.DMA,
            pltpu.SemaphoreType.DMA,
            pltpu.SemaphoreType.DMA,
        ],
    )
    return f(idx, pos_table, token_table)


def kernel(input_ids, token_table, pos_table):
    idx = input_ids.reshape(NW, NCHUNK, CHUNK).astype(jnp.int32)
    out = _emb(idx, token_table, pos_table)
    return out.reshape(BATCH, SEQ, HIDDEN)
